# Initial kernel scaffold; baseline (speedup 1.0000x reference)
import jax
import jax.numpy as jnp
from jax.experimental import pallas as pl


def kernel(x, edge_index, edge_type, W_q, W_k, W_t, type_emb, a):
    raise NotImplementedError("write your pallas kernel here")


# speedup vs baseline: 56.8871x; 56.8871x over previous
import functools

import jax
import jax.numpy as jnp
from jax import lax
from jax.experimental import pallas as pl
from jax.experimental.pallas import tpu as pltpu
from jax.experimental.pallas import tpu_sc as plsc

_N = 10000
_E = 160000
_H = 512
_NEG_SLOPE = 0.2
_MIN_NORM = 1e-15

_NT = 32
_EPAD = 160256
_EPW = _EPAD // _NT
_VECS = _EPW // 16
_NPAD = 10240
_ZCH = _NPAD // 16


def _node_stage(x_ref, wq_ref, wk_ref, wt_ref, temb_ref, a_ref, s_ref, stb_ref):
    a2 = a_ref[...]
    aq = a2[:, 0:_H]
    ak = a2[:, _H:2 * _H]
    at = a2[:, 2 * _H:3 * _H]
    dn_c0 = (((1,), (0,)), ((), ()))
    dn_c1 = (((1,), (1,)), ((), ()))
    vq = lax.dot_general(aq, wq_ref[...], dn_c0)
    vk = lax.dot_general(ak, wk_ref[...], dn_c0)
    v2 = jnp.concatenate([vq, vk], axis=0)
    s = lax.dot_general(x_ref[...], v2, dn_c1)
    s_ref[...] = s
    tvec = lax.dot_general(at, wt_ref[...], dn_c0)
    st = lax.dot_general(tvec, temb_ref[...], dn_c1)
    bound = (jnp.max(s[:, 0:1]) + jnp.max(s[:, 1:2]) + jnp.max(st))
    stb_ref[0:1, :] = st
    stb_ref[1:2, :] = jnp.full((1, 16), bound, jnp.float32)


def _node_call(x, W_q, W_k, W_t, type_emb, a2):
    return pl.pallas_call(
        _node_stage,
        out_shape=[
            jax.ShapeDtypeStruct((_N, 2), jnp.float32),
            jax.ShapeDtypeStruct((2, 16), jnp.float32),
        ],
    )(x, W_q, W_k, W_t, type_emb, a2)


def _edge_a_body(sq_hbm, sk_hbm, stb_hbm, src_hbm, dst_hbm, et_hbm,
                 e_hbm, part_hbm,
                 src_v, dst_v, et_v, sq_v, sk_v, stb_v, e_v, den_sh, sem):
    c = lax.axis_index("c")
    s = lax.axis_index("s")
    w = c * 16 + s
    base = w * _EPW

    cp = pltpu.make_async_copy
    cps = [
        cp(src_hbm.at[pl.ds(base, _EPW)], src_v, sem),
        cp(dst_hbm.at[pl.ds(base, _EPW)], dst_v, sem),
        cp(et_hbm.at[pl.ds(base, _EPW)], et_v, sem),
        cp(sq_hbm, sq_v.at[pl.ds(0, _N)], sem),
        cp(sk_hbm, sk_v.at[pl.ds(0, _N)], sem),
        cp(stb_hbm, stb_v, sem),
    ]
    for c_ in cps:
        c_.start()

    for i in range(15):
        sq_v[pl.ds(_N + i * 16, 16)] = jnp.zeros((16,), jnp.float32)
    for i in range(_ZCH // 16):
        e_v[pl.ds(i * 16, 16)] = jnp.zeros((16,), jnp.float32)
    pltpu.sync_copy(e_v.at[pl.ds(0, _ZCH)], den_sh.at[pl.ds(s * _ZCH, _ZCH)])

    for c_ in cps:
        c_.wait()

    bvec = stb_v[pl.ds(16, 16)]

    def ebody(i, carry):
        sl = pl.ds(i * 16, 16)
        di = dst_v[sl]
        si = src_v[sl]
        ti = et_v[sl]
        z = (plsc.load_gather(sq_v, [di])
             + plsc.load_gather(sk_v, [si])
             + plsc.load_gather(stb_v, [ti]))
        raw = jnp.where(z >= 0.0, z, z * _NEG_SLOPE)
        e_v[sl] = jnp.exp(raw - bvec)
        return carry

    lax.fori_loop(0, _VECS, ebody, 0)

    eout = cp(e_v, e_hbm.at[pl.ds(base, _EPW)], sem)
    eout.start()

    plsc.subcore_barrier()
    pltpu.sync_copy(e_v, den_sh.at[dst_v], add=True)
    plsc.subcore_barrier()

    pltpu.sync_copy(den_sh.at[pl.ds(s * _ZCH, _ZCH)],
                    part_hbm.at[c, pl.ds(s * _ZCH, _ZCH)])
    eout.wait()


def _edge_b_body(e_hbm, dst_hbm, part_hbm, out_hbm,
                 dst_v, e_v, d0_v, d1_v, den_v, den_sh, sem):
    c = lax.axis_index("c")
    s = lax.axis_index("s")
    w = c * 16 + s
    base = w * _EPW

    cp = pltpu.make_async_copy
    cps = [
        cp(dst_hbm.at[pl.ds(base, _EPW)], dst_v, sem),
        cp(e_hbm.at[pl.ds(base, _EPW)], e_v, sem),
        cp(part_hbm.at[0, pl.ds(s * _ZCH, _ZCH)], d0_v, sem),
        cp(part_hbm.at[1, pl.ds(s * _ZCH, _ZCH)], d1_v, sem),
    ]
    for c_ in cps:
        c_.start()
    for c_ in cps:
        c_.wait()

    for i in range(_ZCH // 16):
        sl = pl.ds(i * 16, 16)
        d0_v[sl] = jnp.maximum(d0_v[sl] + d1_v[sl], _MIN_NORM)
    pltpu.sync_copy(d0_v, den_sh.at[pl.ds(s * _ZCH, _ZCH)])
    plsc.subcore_barrier()
    pltpu.sync_copy(den_sh, den_v)

    def nbody(i, carry):
        sl = pl.ds(i * 16, 16)
        di = dst_v[sl]
        dn = plsc.load_gather(den_v, [di])
        e_v[sl] = e_v[sl] / dn
        return carry

    lax.fori_loop(0, _VECS, nbody, 0)

    pltpu.sync_copy(e_v, out_hbm.at[pl.ds(base, _EPW)])


def _make_edge_calls():
    mesh = plsc.VectorSubcoreMesh(core_axis_name="c", subcore_axis_name="s",
                                  num_cores=2)
    params = pltpu.CompilerParams(needs_layout_passes=False)
    call_a = functools.partial(
        pl.kernel,
        mesh=mesh,
        compiler_params=params,
        out_type=[
            jax.ShapeDtypeStruct((_EPAD,), jnp.float32),
            jax.ShapeDtypeStruct((2, _NPAD), jnp.float32),
        ],
        scratch_types=[
            pltpu.VMEM((_EPW,), jnp.int32),
            pltpu.VMEM((_EPW,), jnp.int32),
            pltpu.VMEM((_EPW,), jnp.int32),
            pltpu.VMEM((_NPAD,), jnp.float32),
            pltpu.VMEM((_N,), jnp.float32),
            pltpu.VMEM((32,), jnp.float32),
            pltpu.VMEM((_EPW,), jnp.float32),
            pltpu.VMEM_SHARED((_NPAD,), jnp.float32),
            pltpu.SemaphoreType.DMA,
        ],
    )(_edge_a_body)
    call_b = functools.partial(
        pl.kernel,
        mesh=mesh,
        compiler_params=params,
        out_type=jax.ShapeDtypeStruct((_EPAD,), jnp.float32),
        scratch_types=[
            pltpu.VMEM((_EPW,), jnp.int32),
            pltpu.VMEM((_EPW,), jnp.float32),
            pltpu.VMEM((_ZCH,), jnp.float32),
            pltpu.VMEM((_ZCH,), jnp.float32),
            pltpu.VMEM((_NPAD,), jnp.float32),
            pltpu.VMEM_SHARED((_NPAD,), jnp.float32),
            pltpu.SemaphoreType.DMA,
        ],
    )(_edge_b_body)
    return call_a, call_b


def kernel(x, edge_index, edge_type, W_q, W_k, W_t, type_emb, a):
    a2 = a.reshape(1, 3 * _H)
    s, stb = _node_call(x, W_q, W_k, W_t, type_emb, a2)
    sq = s[:, 0]
    sk = s[:, 1]
    stb_flat = stb.reshape(32)
    npad = _EPAD - _E
    src = jnp.concatenate([edge_index[0], jnp.zeros((npad,), jnp.int32)])
    dst = jnp.concatenate([edge_index[1], jnp.full((npad,), _N, jnp.int32)])
    et = jnp.concatenate([edge_type, jnp.zeros((npad,), jnp.int32)])
    call_a, call_b = _make_edge_calls()
    e_all, part = call_a(sq, sk, stb_flat, src, dst, et)
    alpha_pad = call_b(e_all, dst, part)
    return alpha_pad[:_E]


# speedup vs baseline: 63.4575x; 1.1155x over previous
import functools

import jax
import jax.numpy as jnp
from jax import lax
from jax.experimental import pallas as pl
from jax.experimental.pallas import tpu as pltpu
from jax.experimental.pallas import tpu_sc as plsc

_N = 10000
_E = 160000
_H = 512
_NEG_SLOPE = 0.2
_MIN_NORM = 1e-15

_NT = 32
_EPAD = 160256
_EPW = _EPAD // _NT
_VECS = _EPW // 16
_NPAD = 10240
_ZCH = _NPAD // 16


def _node_stage(x_ref, wq_ref, wk_ref, wt_ref, temb_ref, a_ref, s_ref, stb_ref):
    a2 = a_ref[...]
    aq = a2[:, 0:_H]
    ak = a2[:, _H:2 * _H]
    at = a2[:, 2 * _H:3 * _H]
    dn_c0 = (((1,), (0,)), ((), ()))
    dn_c1 = (((1,), (1,)), ((), ()))
    vq = lax.dot_general(aq, wq_ref[...], dn_c0)
    vk = lax.dot_general(ak, wk_ref[...], dn_c0)
    v2 = jnp.concatenate([vq, vk], axis=0)
    s = lax.dot_general(x_ref[...], v2, dn_c1)
    s_ref[...] = s
    tvec = lax.dot_general(at, wt_ref[...], dn_c0)
    st = lax.dot_general(tvec, temb_ref[...], dn_c1)
    bound = (jnp.max(s[:, 0:1]) + jnp.max(s[:, 1:2]) + jnp.max(st))
    stb_ref[0:1, :] = st
    stb_ref[1:2, :] = jnp.full((1, 16), bound, jnp.float32)


def _node_call(x, W_q, W_k, W_t, type_emb, a2):
    return pl.pallas_call(
        _node_stage,
        out_shape=[
            jax.ShapeDtypeStruct((_N, 2), jnp.float32),
            jax.ShapeDtypeStruct((2, 16), jnp.float32),
        ],
    )(x, W_q, W_k, W_t, type_emb, a2)


def _edge_a_body(sq_hbm, sk_hbm, stb_hbm, src_hbm, dst_hbm, et_hbm,
                 e_hbm, part_hbm,
                 src_v, dst_v, et_v, sq_v, sk_v, stb_v, e_v, den_sh, sem):
    c = lax.axis_index("c")
    s = lax.axis_index("s")
    w = c * 16 + s
    base = w * _EPW

    cp = pltpu.make_async_copy
    cps = [
        cp(src_hbm.at[pl.ds(base, _EPW)], src_v, sem),
        cp(dst_hbm.at[pl.ds(base, _EPW)], dst_v, sem),
        cp(et_hbm.at[pl.ds(base, _EPW)], et_v, sem),
        cp(sq_hbm, sq_v.at[pl.ds(0, _N)], sem),
        cp(sk_hbm, sk_v.at[pl.ds(0, _N)], sem),
        cp(stb_hbm, stb_v, sem),
    ]
    for c_ in cps:
        c_.start()

    for i in range(15):
        sq_v[pl.ds(_N + i * 16, 16)] = jnp.zeros((16,), jnp.float32)
    for i in range(_ZCH // 16):
        e_v[pl.ds(i * 16, 16)] = jnp.zeros((16,), jnp.float32)
    pltpu.sync_copy(e_v.at[pl.ds(0, _ZCH)], den_sh.at[pl.ds(s * _ZCH, _ZCH)])

    for c_ in cps:
        c_.wait()

    bvec = stb_v[pl.ds(16, 16)]

    @plsc.parallel_loop(0, _EPW, 16, unroll=8)
    def _ebody(i):
        sl = pl.ds(i, 16)
        di = dst_v[sl]
        si = src_v[sl]
        ti = et_v[sl]
        z = (plsc.load_gather(sq_v, [di])
             + plsc.load_gather(sk_v, [si])
             + plsc.load_gather(stb_v, [ti]))
        raw = jnp.where(z >= 0.0, z, z * _NEG_SLOPE)
        e_v[sl] = jnp.exp(raw - bvec)

    eout = cp(e_v, e_hbm.at[pl.ds(base, _EPW)], sem)
    eout.start()

    plsc.subcore_barrier()
    pltpu.sync_copy(e_v, den_sh.at[dst_v], add=True)
    plsc.subcore_barrier()

    pltpu.sync_copy(den_sh.at[pl.ds(s * _ZCH, _ZCH)],
                    part_hbm.at[c, pl.ds(s * _ZCH, _ZCH)])
    eout.wait()


def _edge_b_body(e_hbm, dst_hbm, part_hbm, out_hbm,
                 dst_v, e_v, d0_v, d1_v, den_v, den_sh, sem):
    c = lax.axis_index("c")
    s = lax.axis_index("s")
    w = c * 16 + s
    base = w * _EPW

    cp = pltpu.make_async_copy
    cps = [
        cp(dst_hbm.at[pl.ds(base, _EPW)], dst_v, sem),
        cp(e_hbm.at[pl.ds(base, _EPW)], e_v, sem),
        cp(part_hbm.at[0, pl.ds(s * _ZCH, _ZCH)], d0_v, sem),
        cp(part_hbm.at[1, pl.ds(s * _ZCH, _ZCH)], d1_v, sem),
    ]
    for c_ in cps:
        c_.start()
    for c_ in cps:
        c_.wait()

    for i in range(_ZCH // 16):
        sl = pl.ds(i * 16, 16)
        d0_v[sl] = jnp.maximum(d0_v[sl] + d1_v[sl], _MIN_NORM)
    pltpu.sync_copy(d0_v, den_sh.at[pl.ds(s * _ZCH, _ZCH)])
    plsc.subcore_barrier()
    pltpu.sync_copy(den_sh, den_v)

    @plsc.parallel_loop(0, _EPW, 16, unroll=8)
    def _nbody(i):
        sl = pl.ds(i, 16)
        di = dst_v[sl]
        dn = plsc.load_gather(den_v, [di])
        e_v[sl] = e_v[sl] / dn

    pltpu.sync_copy(e_v, out_hbm.at[pl.ds(base, _EPW)])


def _make_edge_calls():
    mesh = plsc.VectorSubcoreMesh(core_axis_name="c", subcore_axis_name="s",
                                  num_cores=2)
    params = pltpu.CompilerParams(needs_layout_passes=False)
    call_a = functools.partial(
        pl.kernel,
        mesh=mesh,
        compiler_params=params,
        out_type=[
            jax.ShapeDtypeStruct((_EPAD,), jnp.float32),
            jax.ShapeDtypeStruct((2, _NPAD), jnp.float32),
        ],
        scratch_types=[
            pltpu.VMEM((_EPW,), jnp.int32),
            pltpu.VMEM((_EPW,), jnp.int32),
            pltpu.VMEM((_EPW,), jnp.int32),
            pltpu.VMEM((_NPAD,), jnp.float32),
            pltpu.VMEM((_N,), jnp.float32),
            pltpu.VMEM((32,), jnp.float32),
            pltpu.VMEM((_EPW,), jnp.float32),
            pltpu.VMEM_SHARED((_NPAD,), jnp.float32),
            pltpu.SemaphoreType.DMA,
        ],
    )(_edge_a_body)
    call_b = functools.partial(
        pl.kernel,
        mesh=mesh,
        compiler_params=params,
        out_type=jax.ShapeDtypeStruct((_EPAD,), jnp.float32),
        scratch_types=[
            pltpu.VMEM((_EPW,), jnp.int32),
            pltpu.VMEM((_EPW,), jnp.float32),
            pltpu.VMEM((_ZCH,), jnp.float32),
            pltpu.VMEM((_ZCH,), jnp.float32),
            pltpu.VMEM((_NPAD,), jnp.float32),
            pltpu.VMEM_SHARED((_NPAD,), jnp.float32),
            pltpu.SemaphoreType.DMA,
        ],
    )(_edge_b_body)
    return call_a, call_b


def kernel(x, edge_index, edge_type, W_q, W_k, W_t, type_emb, a):
    a2 = a.reshape(1, 3 * _H)
    s, stb = _node_call(x, W_q, W_k, W_t, type_emb, a2)
    sq = s[:, 0]
    sk = s[:, 1]
    stb_flat = stb.reshape(32)
    npad = _EPAD - _E
    src = jnp.concatenate([edge_index[0], jnp.zeros((npad,), jnp.int32)])
    dst = jnp.concatenate([edge_index[1], jnp.full((npad,), _N, jnp.int32)])
    et = jnp.concatenate([edge_type, jnp.zeros((npad,), jnp.int32)])
    call_a, call_b = _make_edge_calls()
    e_all, part = call_a(sq, sk, stb_flat, src, dst, et)
    alpha_pad = call_b(e_all, dst, part)
    return alpha_pad[:_E]


# speedup vs baseline: 75.7969x; 1.1945x over previous
import functools

import jax
import jax.numpy as jnp
from jax import lax
from jax.experimental import pallas as pl
from jax.experimental.pallas import tpu as pltpu
from jax.experimental.pallas import tpu_sc as plsc

_N = 10000
_E = 160000
_H = 512
_NEG_SLOPE = 0.2
_MIN_NORM = 1e-15

_NW = 16
_EPW = _E // _NW
_NPAD = 10240
_ZCH = _NPAD // _NW


def _node_stage(x_ref, wq_ref, wk_ref, wt_ref, temb_ref, a_ref, s_ref, stb_ref):
    a2 = a_ref[...]
    aq = a2[:, 0:_H]
    ak = a2[:, _H:2 * _H]
    at = a2[:, 2 * _H:3 * _H]
    dn_c0 = (((1,), (0,)), ((), ()))
    dn_c1 = (((1,), (1,)), ((), ()))
    vq = lax.dot_general(aq, wq_ref[...], dn_c0)
    vk = lax.dot_general(ak, wk_ref[...], dn_c0)
    v2 = jnp.concatenate([vq, vk], axis=0)
    s = lax.dot_general(x_ref[...], v2, dn_c1)
    s_ref[...] = s
    tvec = lax.dot_general(at, wt_ref[...], dn_c0)
    st = lax.dot_general(tvec, temb_ref[...], dn_c1)
    bound = (jnp.max(s[:, 0:1]) + jnp.max(s[:, 1:2]) + jnp.max(st))
    stb_ref[0:1, :] = st
    stb_ref[1:2, :] = jnp.full((1, 16), bound, jnp.float32)


def _node_call(x, W_q, W_k, W_t, type_emb, a2):
    return pl.pallas_call(
        _node_stage,
        out_shape=[
            jax.ShapeDtypeStruct((_N, 2), jnp.float32),
            jax.ShapeDtypeStruct((2, 16), jnp.float32),
        ],
    )(x, W_q, W_k, W_t, type_emb, a2)


def _edge_body(s_hbm, stb_hbm, src_hbm, dst_hbm, et_hbm, out_hbm,
               src_v, dst_v, et_v, s_v, stb_v, e_v, den_v, den_sh, sem):
    w = lax.axis_index("s")
    base = w * _EPW

    cp = pltpu.make_async_copy
    cps = [
        cp(src_hbm.at[pl.ds(base, _EPW)], src_v, sem),
        cp(dst_hbm.at[pl.ds(base, _EPW)], dst_v, sem),
        cp(et_hbm.at[pl.ds(base, _EPW)], et_v, sem),
        cp(s_hbm, s_v, sem),
        cp(stb_hbm, stb_v, sem),
    ]
    for c_ in cps:
        c_.start()

    for i in range(_ZCH // 16):
        e_v[pl.ds(i * 16, 16)] = jnp.zeros((16,), jnp.float32)
    pltpu.sync_copy(e_v.at[pl.ds(0, _ZCH)], den_sh.at[pl.ds(w * _ZCH, _ZCH)])

    for c_ in cps:
        c_.wait()

    bvec = stb_v[pl.ds(16, 16)]
    one = jnp.full((16,), 1, jnp.int32)

    @plsc.parallel_loop(0, _EPW, 16, unroll=8)
    def _ebody(i):
        sl = pl.ds(i, 16)
        di = dst_v[sl]
        si = src_v[sl]
        ti = et_v[sl]
        z = (plsc.load_gather(s_v, [di + di])
             + plsc.load_gather(s_v, [si + si + one])
             + plsc.load_gather(stb_v, [ti]))
        raw = jnp.where(z >= 0.0, z, z * _NEG_SLOPE)
        e_v[sl] = jnp.exp(raw - bvec)

    plsc.subcore_barrier()
    pltpu.sync_copy(e_v, den_sh.at[dst_v], add=True)
    plsc.subcore_barrier()

    pltpu.sync_copy(den_sh, den_v)

    @plsc.parallel_loop(0, _EPW, 16, unroll=8)
    def _nbody(i):
        sl = pl.ds(i, 16)
        di = dst_v[sl]
        dn = jnp.maximum(plsc.load_gather(den_v, [di]), _MIN_NORM)
        e_v[sl] = e_v[sl] / dn

    pltpu.sync_copy(e_v, out_hbm.at[pl.ds(base, _EPW)])


def _make_edge_call():
    mesh = plsc.VectorSubcoreMesh(core_axis_name="c", subcore_axis_name="s",
                                  num_cores=1)
    return functools.partial(
        pl.kernel,
        mesh=mesh,
        compiler_params=pltpu.CompilerParams(needs_layout_passes=False),
        out_type=jax.ShapeDtypeStruct((_E,), jnp.float32),
        scratch_types=[
            pltpu.VMEM((_EPW,), jnp.int32),
            pltpu.VMEM((_EPW,), jnp.int32),
            pltpu.VMEM((_EPW,), jnp.int32),
            pltpu.VMEM((2 * _N,), jnp.float32),
            pltpu.VMEM((32,), jnp.float32),
            pltpu.VMEM((_EPW,), jnp.float32),
            pltpu.VMEM((_NPAD,), jnp.float32),
            pltpu.VMEM_SHARED((_NPAD,), jnp.float32),
            pltpu.SemaphoreType.DMA,
        ],
    )(_edge_body)


def kernel(x, edge_index, edge_type, W_q, W_k, W_t, type_emb, a):
    a2 = a.reshape(1, 3 * _H)
    s, stb = _node_call(x, W_q, W_k, W_t, type_emb, a2)
    edge_call = _make_edge_call()
    return edge_call(s.reshape(2 * _N), stb.reshape(32),
                     edge_index[0], edge_index[1], edge_type)


# speedup vs baseline: 95.7709x; 1.2635x over previous
import functools

import jax
import jax.numpy as jnp
from jax import lax
from jax.experimental import pallas as pl
from jax.experimental.pallas import tpu as pltpu
from jax.experimental.pallas import tpu_sc as plsc

_N = 10000
_E = 160000
_H = 512
_NEG_SLOPE = 0.2
_MIN_NORM = 1e-15

_NW = 16
_EPW = _E // _NW
_NPAD = 10240
_ZCH = _NPAD // _NW


def _node_stage(x_ref, wq_ref, wk_ref, wt_ref, temb_ref, a_ref, ei_ref,
                s_ref, stb_ref, src_ref, dst_ref):
    a2 = a_ref[...]
    aq = a2[:, 0:_H]
    ak = a2[:, _H:2 * _H]
    at = a2[:, 2 * _H:3 * _H]
    dn_c0 = (((1,), (0,)), ((), ()))
    dn_c1 = (((1,), (1,)), ((), ()))
    vq = lax.dot_general(aq, wq_ref[...], dn_c0)
    vk = lax.dot_general(ak, wk_ref[...], dn_c0)
    v2 = jnp.concatenate([vq, vk], axis=0)
    s = lax.dot_general(v2, x_ref[...], dn_c1)
    s_ref[...] = s
    tvec = lax.dot_general(at, wt_ref[...], dn_c0)
    st = lax.dot_general(tvec, temb_ref[...], dn_c1)
    bound = (jnp.max(s[0:1, :]) + jnp.max(s[1:2, :]) + jnp.max(st))
    stb_ref[0:1, :] = st
    stb_ref[1:2, :] = jnp.full((1, 16), bound, jnp.float32)
    src_ref[...] = ei_ref[0, :]
    dst_ref[...] = ei_ref[1, :]


def _node_call(x, W_q, W_k, W_t, type_emb, a2, edge_index):
    return pl.pallas_call(
        _node_stage,
        out_shape=[
            jax.ShapeDtypeStruct((2, _N), jnp.float32),
            jax.ShapeDtypeStruct((2, 16), jnp.float32),
            jax.ShapeDtypeStruct((_E,), jnp.int32),
            jax.ShapeDtypeStruct((_E,), jnp.int32),
        ],
    )(x, W_q, W_k, W_t, type_emb, a2, edge_index)


def _edge_body(s_hbm, stb_hbm, src_hbm, dst_hbm, et_hbm, out_hbm,
               src_v, dst_v, et_v, s_v, stb_v, e_v, den_v, den_sh, sem):
    w = lax.axis_index("s")
    base = w * _EPW

    cp = pltpu.make_async_copy
    cps = [
        cp(src_hbm.at[pl.ds(base, _EPW)], src_v, sem),
        cp(dst_hbm.at[pl.ds(base, _EPW)], dst_v, sem),
        cp(et_hbm.at[pl.ds(base, _EPW)], et_v, sem),
        cp(s_hbm, s_v, sem),
        cp(stb_hbm, stb_v, sem),
    ]
    for c_ in cps:
        c_.start()

    for i in range(_ZCH // 16):
        e_v[pl.ds(i * 16, 16)] = jnp.zeros((16,), jnp.float32)
    pltpu.sync_copy(e_v.at[pl.ds(0, _ZCH)], den_sh.at[pl.ds(w * _ZCH, _ZCH)])

    for c_ in cps:
        c_.wait()

    bvec = stb_v[pl.ds(16, 16)]
    noff = jnp.full((16,), _N, jnp.int32)

    @plsc.parallel_loop(0, _EPW, 16, unroll=8)
    def _ebody(i):
        sl = pl.ds(i, 16)
        di = dst_v[sl]
        si = src_v[sl]
        ti = et_v[sl]
        z = (plsc.load_gather(s_v, [di])
             + plsc.load_gather(s_v, [si + noff])
             + plsc.load_gather(stb_v, [ti]))
        raw = jnp.where(z >= 0.0, z, z * _NEG_SLOPE)
        e_v[sl] = jnp.exp(raw - bvec)

    plsc.subcore_barrier()
    pltpu.sync_copy(e_v, den_sh.at[dst_v], add=True)
    plsc.subcore_barrier()

    pltpu.sync_copy(den_sh, den_v)

    @plsc.parallel_loop(0, _EPW, 16, unroll=8)
    def _nbody(i):
        sl = pl.ds(i, 16)
        di = dst_v[sl]
        dn = jnp.maximum(plsc.load_gather(den_v, [di]), _MIN_NORM)
        e_v[sl] = e_v[sl] / dn

    pltpu.sync_copy(e_v, out_hbm.at[pl.ds(base, _EPW)])


def _make_edge_call():
    mesh = plsc.VectorSubcoreMesh(core_axis_name="c", subcore_axis_name="s",
                                  num_cores=1)
    return functools.partial(
        pl.kernel,
        mesh=mesh,
        compiler_params=pltpu.CompilerParams(needs_layout_passes=False),
        out_type=jax.ShapeDtypeStruct((_E,), jnp.float32),
        scratch_types=[
            pltpu.VMEM((_EPW,), jnp.int32),
            pltpu.VMEM((_EPW,), jnp.int32),
            pltpu.VMEM((_EPW,), jnp.int32),
            pltpu.VMEM((2 * _N,), jnp.float32),
            pltpu.VMEM((32,), jnp.float32),
            pltpu.VMEM((_EPW,), jnp.float32),
            pltpu.VMEM((_NPAD,), jnp.float32),
            pltpu.VMEM_SHARED((_NPAD,), jnp.float32),
            pltpu.SemaphoreType.DMA,
        ],
    )(_edge_body)


def kernel(x, edge_index, edge_type, W_q, W_k, W_t, type_emb, a):
    a2 = a.reshape(1, 3 * _H)
    s, stb, src, dst = _node_call(x, W_q, W_k, W_t, type_emb, a2, edge_index)
    edge_call = _make_edge_call()
    return edge_call(s.reshape(2 * _N), stb.reshape(32), src, dst, edge_type)


# speedup vs baseline: 98.6247x; 1.0298x over previous
import functools

import jax
import jax.numpy as jnp
from jax import lax
from jax.experimental import pallas as pl
from jax.experimental.pallas import tpu as pltpu
from jax.experimental.pallas import tpu_sc as plsc

_N = 10000
_E = 160000
_H = 512
_NEG_SLOPE = 0.2
_MIN_NORM = 1e-15

_NW = 16
_EPW = _E // _NW
_NPAD = 10240
_ZCH = _NPAD // _NW


def _node_stage(x_ref, wq_ref, wk_ref, wt_ref, temb_ref, a_ref, ei_ref,
                s_ref, stb_ref, src_ref, dst_ref):
    a2 = a_ref[...]
    aq = a2[:, 0:_H]
    ak = a2[:, _H:2 * _H]
    at = a2[:, 2 * _H:3 * _H]
    dn_c0 = (((1,), (0,)), ((), ()))
    dn_c1 = (((1,), (1,)), ((), ()))
    vq = lax.dot_general(aq, wq_ref[...], dn_c0)
    vk = lax.dot_general(ak, wk_ref[...], dn_c0)
    v2 = jnp.concatenate([vq, vk], axis=0)
    s = lax.dot_general(v2, x_ref[...], dn_c1)
    s_ref[...] = s
    tvec = lax.dot_general(at, wt_ref[...], dn_c0)
    st = lax.dot_general(tvec, temb_ref[...], dn_c1)
    bound = 2.0 * jnp.max(s) + jnp.max(st)
    stb_ref[0:1, :] = st
    stb_ref[1:2, :] = jnp.full((1, 16), bound, jnp.float32)
    src_ref[...] = ei_ref[0, :]
    dst_ref[...] = ei_ref[1, :]


def _node_call(x, W_q, W_k, W_t, type_emb, a2, edge_index):
    return pl.pallas_call(
        _node_stage,
        out_shape=[
            jax.ShapeDtypeStruct((2, _N), jnp.float32),
            jax.ShapeDtypeStruct((2, 16), jnp.float32),
            jax.ShapeDtypeStruct((_E,), jnp.int32),
            jax.ShapeDtypeStruct((_E,), jnp.int32),
        ],
    )(x, W_q, W_k, W_t, type_emb, a2, edge_index)


_CHUNKS = (2560, 2560, 2560, 2320)


def _edge_body(s_hbm, stb_hbm, src_hbm, dst_hbm, et_hbm, out_hbm,
               src_v, d0_v, d1_v, d2_v, d3_v, et_v, s_v, stb_v, e_v, den_v,
               den_sh, sem, sem2):
    w = lax.axis_index("s")
    base = w * _EPW
    dchunks = (d0_v, d1_v, d2_v, d3_v)

    cp = pltpu.make_async_copy
    cps = [
        cp(src_hbm.at[pl.ds(base, _EPW)], src_v, sem),
        cp(et_hbm.at[pl.ds(base, _EPW)], et_v, sem),
        cp(s_hbm, s_v, sem),
        cp(stb_hbm, stb_v, sem),
    ]
    off = 0
    for k, ch in enumerate(_CHUNKS):
        cps.append(cp(dst_hbm.at[pl.ds(base + off, ch)], dchunks[k], sem))
        off += ch
    for c_ in cps:
        c_.start()

    for i in range(_ZCH // 16):
        e_v[pl.ds(i * 16, 16)] = jnp.zeros((16,), jnp.float32)
    pltpu.sync_copy(e_v.at[pl.ds(0, _ZCH)], den_sh.at[pl.ds(w * _ZCH, _ZCH)])

    for c_ in cps:
        c_.wait()

    plsc.subcore_barrier()

    bvec = stb_v[pl.ds(16, 16)]
    noff = jnp.full((16,), _N, jnp.int32)

    scs = []
    off = 0
    for k, ch in enumerate(_CHUNKS):
        dck = dchunks[k]
        eoff = off

        @plsc.parallel_loop(0, ch, 16, unroll=8)
        def _ebody(i, _dck=dck, _eoff=eoff):
            di = _dck[pl.ds(i, 16)]
            sl = pl.ds(i + _eoff, 16)
            si = src_v[sl]
            ti = et_v[sl]
            z = (plsc.load_gather(s_v, [di])
                 + plsc.load_gather(s_v, [si + noff])
                 + plsc.load_gather(stb_v, [ti]))
            raw = jnp.where(z >= 0.0, z, z * _NEG_SLOPE)
            e_v[sl] = jnp.exp(raw - bvec)

        scs.append(pltpu.async_copy(e_v.at[pl.ds(off, ch)],
                                    den_sh.at[dck], sem2, add=True))
        off += ch

    for c_ in scs:
        c_.wait()
    plsc.subcore_barrier()

    pltpu.sync_copy(den_sh, den_v)

    off = 0
    for k, ch in enumerate(_CHUNKS):
        dck = dchunks[k]
        eoff = off

        @plsc.parallel_loop(0, ch, 16, unroll=8)
        def _nbody(i, _dck=dck, _eoff=eoff):
            di = _dck[pl.ds(i, 16)]
            sl = pl.ds(i + _eoff, 16)
            dn = jnp.maximum(plsc.load_gather(den_v, [di]), _MIN_NORM)
            e_v[sl] = e_v[sl] / dn

        off += ch

    pltpu.sync_copy(e_v, out_hbm.at[pl.ds(base, _EPW)])


def _make_edge_call():
    mesh = plsc.VectorSubcoreMesh(core_axis_name="c", subcore_axis_name="s",
                                  num_cores=1)
    return functools.partial(
        pl.kernel,
        mesh=mesh,
        compiler_params=pltpu.CompilerParams(needs_layout_passes=False),
        out_type=jax.ShapeDtypeStruct((_E,), jnp.float32),
        scratch_types=[
            pltpu.VMEM((_EPW,), jnp.int32),
            pltpu.VMEM((_CHUNKS[0],), jnp.int32),
            pltpu.VMEM((_CHUNKS[1],), jnp.int32),
            pltpu.VMEM((_CHUNKS[2],), jnp.int32),
            pltpu.VMEM((_CHUNKS[3],), jnp.int32),
            pltpu.VMEM((_EPW,), jnp.int32),
            pltpu.VMEM((2 * _N,), jnp.float32),
            pltpu.VMEM((32,), jnp.float32),
            pltpu.VMEM((_EPW,), jnp.float32),
            pltpu.VMEM((_NPAD,), jnp.float32),
            pltpu.VMEM_SHARED((_NPAD,), jnp.float32),
            pltpu.SemaphoreType.DMA,
            pltpu.SemaphoreType.DMA,
        ],
    )(_edge_body)


def kernel(x, edge_index, edge_type, W_q, W_k, W_t, type_emb, a):
    a2 = a.reshape(1, 3 * _H)
    s, stb, src, dst = _node_call(x, W_q, W_k, W_t, type_emb, a2, edge_index)
    edge_call = _make_edge_call()
    return edge_call(s.reshape(2 * _N), stb.reshape(32), src, dst, edge_type)
